# Initial kernel scaffold; baseline (speedup 1.0000x reference)
import jax
import jax.numpy as jnp
from jax.experimental import pallas as pl


def kernel(data, ivectors):
    raise NotImplementedError("write your pallas kernel here")


# speedup vs baseline: 1.7976x; 1.7976x over previous
import functools

import jax
import jax.numpy as jnp
from jax import lax
from jax.experimental import pallas as pl
from jax.experimental.pallas import tpu as pltpu
from jax.experimental.pallas import tpu_sc as plsc

_D = 64
_NW = 32
_C = 512


def _gather_rows(idx, table):
    B = idx.shape[0]
    b_per_w = B // _NW
    n_chunks = b_per_w // _C

    mesh = plsc.VectorSubcoreMesh(core_axis_name="c", subcore_axis_name="s")

    @functools.partial(
        pl.kernel,
        mesh=mesh,
        out_type=jax.ShapeDtypeStruct((B, _D), jnp.float32),
        compiler_params=pltpu.CompilerParams(use_tc_tiling_on_sc=False),
        scratch_types=[
            pltpu.VMEM((_C,), jnp.int32),
            pltpu.VMEM((_C, _D), jnp.float32),
            pltpu.SemaphoreType.DMA,
        ],
    )
    def k(idx_hbm, table_hbm, out_hbm, idx_v, rows_v, sem):
        wid = lax.axis_index("s") * 2 + lax.axis_index("c")
        base = wid * b_per_w

        def body(g, carry):
            off = pl.multiple_of(base + g * _C, _C)
            pltpu.sync_copy(idx_hbm.at[pl.ds(off, _C)], idx_v)
            pltpu.async_copy(table_hbm.at[idx_v], rows_v, sem).wait()
            pltpu.sync_copy(rows_v, out_hbm.at[pl.ds(off, _C)])
            return carry

        lax.fori_loop(0, n_chunks, body, 0)

    return k(idx, table)


def kernel(data, ivectors):
    B0, T = data.shape
    idx = data.reshape(B0 * T).astype(jnp.int32)
    out = _gather_rows(idx, ivectors)
    return out.reshape(B0, T, _D)


# speedup vs baseline: 1.8700x; 1.0402x over previous
import functools

import jax
import jax.numpy as jnp
from jax import lax
from jax.experimental import pallas as pl
from jax.experimental.pallas import tpu as pltpu
from jax.experimental.pallas import tpu_sc as plsc

_D = 64
_NW = 32
_C = 256
_NBUF = 4


def _gather_rows(idx, table):
    B = idx.shape[0]
    b_per_w = B // _NW
    n_chunks = b_per_w // _C
    n_groups = n_chunks // _NBUF

    mesh = plsc.VectorSubcoreMesh(core_axis_name="c", subcore_axis_name="s")

    @functools.partial(
        pl.kernel,
        mesh=mesh,
        out_type=jax.ShapeDtypeStruct((B, _D), jnp.float32),
        compiler_params=pltpu.CompilerParams(use_tc_tiling_on_sc=False),
        scratch_types=(
            [pltpu.VMEM((b_per_w,), jnp.int32)]
            + [pltpu.VMEM((_C, _D), jnp.float32) for _ in range(_NBUF)]
            + [pltpu.SemaphoreType.DMA for _ in range(2 * _NBUF)]
        ),
    )
    def k(idx_hbm, table_hbm, out_hbm, idx_v, *bufs_and_sems):
        rows = bufs_and_sems[:_NBUF]
        gsem = bufs_and_sems[_NBUF:2 * _NBUF]
        wsem = bufs_and_sems[2 * _NBUF:]

        wid = lax.axis_index("s") * 2 + lax.axis_index("c")
        base = pl.multiple_of(wid * b_per_w, _C)
        pltpu.sync_copy(idx_hbm.at[pl.ds(base, b_per_w)], idx_v)

        def g_start(g, b):
            off = pl.multiple_of(g * _C, _C)
            pltpu.async_copy(table_hbm.at[idx_v.at[pl.ds(off, _C)]],
                             rows[b], gsem[b])

        def g_wait(b):
            pltpu.make_async_copy(table_hbm.at[idx_v.at[pl.ds(0, _C)]],
                                  rows[b], gsem[b]).wait()

        def w_start(g, b):
            off = pl.multiple_of(base + g * _C, _C)
            pltpu.async_copy(rows[b], out_hbm.at[pl.ds(off, _C)], wsem[b])

        def w_wait(b):
            pltpu.make_async_copy(rows[b], out_hbm.at[pl.ds(0, _C)],
                                  wsem[b]).wait()

        for b in range(_NBUF):
            g_start(b, b)

        def body(i, carry):
            g0 = i * _NBUF
            for b in range(_NBUF):
                g_wait(b)
                w_start(g0 + b, b)
            for b in range(_NBUF):
                w_wait(b)
                g_start(g0 + _NBUF + b, b)
            return carry

        lax.fori_loop(0, n_groups - 1, body, 0)

        g0 = (n_groups - 1) * _NBUF
        for b in range(_NBUF):
            g_wait(b)
            w_start(g0 + b, b)
        for b in range(_NBUF):
            w_wait(b)

    return k(idx, table)


def kernel(data, ivectors):
    B0, T = data.shape
    idx = data.reshape(B0 * T).astype(jnp.int32)
    out = _gather_rows(idx, ivectors)
    return out.reshape(B0, T, _D)
